# baseline (speedup 1.0000x reference)
import jax
import jax.numpy as jnp
from jax.experimental import pallas as pl
from jax.experimental.pallas import tpu as pltpu

_DIM = 32
_TILE = 512
_COMMIT = 0.25


def _vq_body(x2_ref, e2_ref, x_ref, emb_ref,
             loss_ref, recon_ref, perp_ref,
             counts_ref, acc_ref, *, n_tokens):
    i = pl.program_id(0)
    nsteps = pl.num_programs(0)
    t = x_ref.shape[0]
    k = emb_ref.shape[0]

    @pl.when(i == 0)
    def _init():
        counts_ref[...] = jnp.zeros_like(counts_ref)
        acc_ref[0, 0] = 0.0

    xt = x_ref[...]
    emb = emb_ref[...]
    m = jax.lax.dot_general(xt, emb, (((1,), (1,)), ((), ())),
                            precision=jax.lax.Precision.DEFAULT)
    d = (x2_ref[...] + e2_ref[...]) - 2.0 * m
    minv = jnp.min(d, axis=1, keepdims=True)
    iota = jax.lax.broadcasted_iota(jnp.int32, (t, k), 1)
    idx = jnp.min(jnp.where(d == minv, iota, k), axis=1)
    onehot = (iota == idx[:, None]).astype(jnp.float32)
    q = jax.lax.dot_general(onehot, emb, (((1,), (0,)), ((), ())),
                            precision=jax.lax.Precision.DEFAULT)
    recon_ref[...] = xt + (q - xt)
    counts_ref[...] += jnp.sum(onehot, axis=0, keepdims=True)
    diff = q - xt
    acc_ref[0, 0] += jnp.sum(diff * diff)

    @pl.when(i == nsteps - 1)
    def _fin():
        mean_sq = acc_ref[0, 0] / float(n_tokens * _DIM)
        loss_ref[0, 0] = mean_sq + _COMMIT * mean_sq
        p = counts_ref[...] * (1.0 / float(n_tokens))
        perp_ref[0, 0] = jnp.exp(-jnp.sum(p * jnp.log(p + 1e-10)))


def kernel(x, embedding):
    input_shape = x.shape
    flat = x.reshape(-1, _DIM)
    n = flat.shape[0]
    k = embedding.shape[0]
    x2 = jnp.sum(flat ** 2, axis=1, keepdims=True)
    e2 = jnp.sum(embedding ** 2, axis=1).reshape(1, k)
    grid = n // _TILE

    import functools
    body = functools.partial(_vq_body, n_tokens=n)
    loss, recon, perp = pl.pallas_call(
        body,
        grid=(grid,),
        in_specs=[
            pl.BlockSpec((_TILE, 1), lambda i: (i, 0)),
            pl.BlockSpec((1, k), lambda i: (0, 0)),
            pl.BlockSpec((_TILE, _DIM), lambda i: (i, 0)),
            pl.BlockSpec((k, _DIM), lambda i: (0, 0)),
        ],
        out_specs=[
            pl.BlockSpec(memory_space=pltpu.SMEM),
            pl.BlockSpec((_TILE, _DIM), lambda i: (i, 0)),
            pl.BlockSpec(memory_space=pltpu.SMEM),
        ],
        out_shape=[
            jax.ShapeDtypeStruct((1, 1), jnp.float32),
            jax.ShapeDtypeStruct((n, _DIM), jnp.float32),
            jax.ShapeDtypeStruct((1, 1), jnp.float32),
        ],
        scratch_shapes=[
            pltpu.VMEM((1, k), jnp.float32),
            pltpu.SMEM((1, 1), jnp.float32),
        ],
        compiler_params=pltpu.CompilerParams(
            dimension_semantics=("arbitrary",)),
    )(x2, e2, flat, embedding)
    return (loss[0, 0], recon.reshape(input_shape), perp[0, 0])
